# baseline (speedup 1.0000x reference)
import functools

import jax
import jax.numpy as jnp
from jax import lax
from jax.experimental import pallas as pl
from jax.experimental.pallas import tpu as pltpu
from jax.experimental.pallas import tpu_sc as plsc

D = 64
N = 4096 * 200
NW = 32
PER_W = N // NW
W = 320
NWIN = PER_W // W
NBUF = 4
SROWS = 192
NG1 = SROWS // 16
DROWS = W - SROWS
NDG = DROWS // 16

_mesh = plsc.VectorSubcoreMesh(core_axis_name="c", subcore_axis_name="s")


@functools.partial(
    pl.kernel,
    mesh=_mesh,
    out_type=jax.ShapeDtypeStruct((N, D), jnp.float32),
    scratch_types=[
        pltpu.VMEM((PER_W,), jnp.int32),
        pltpu.VMEM((NBUF, W, D), jnp.float32),
        [pltpu.SemaphoreType.DMA] * NBUF,
        [pltpu.SemaphoreType.DMA] * NBUF,
        [pltpu.SemaphoreType.DMA] * NBUF,
    ],
    compiler_params=pltpu.CompilerParams(use_tc_tiling_on_sc=False),
)
def _emb_lookup(table_hbm, idx_hbm, out_hbm, idx_all, rows, gsem, dsem, wsem):
    wid = lax.axis_index("s") * 2 + lax.axis_index("c")
    base = wid * PER_W
    pltpu.sync_copy(idx_hbm.at[pl.ds(base, PER_W)], idx_all)

    def fire(g, b):
        for i in range(NG1):
            iv = idx_all[pl.ds(g * W + i * 16, 16)]
            pltpu.async_copy(table_hbm.at[iv], rows.at[b, pl.ds(i * 16, 16)],
                             gsem[b])
        for j in range(NDG):
            sv = idx_all[pl.ds(g * W + SROWS + j * 16, 16)]
            for u in range(16):
                pltpu.async_copy(
                    table_hbm.at[pl.ds(sv[u], 1)],
                    rows.at[b, pl.ds(SROWS + j * 16 + u, 1)], dsem[b])

    def drain(b):
        for i in range(NG1):
            pltpu.make_async_copy(
                table_hbm.at[idx_all[pl.ds(i * 16, 16)]],
                rows.at[b, pl.ds(i * 16, 16)], gsem[b]).wait()
        for k in range(DROWS):
            pltpu.make_async_copy(
                table_hbm.at[pl.ds(0, 1)],
                rows.at[b, pl.ds(SROWS + k, 1)], dsem[b]).wait()

    def start_write(g, b):
        return pltpu.async_copy(
            rows.at[b], out_hbm.at[pl.ds(base + g * W, W)], wsem[b])

    def wait_write(b):
        pltpu.make_async_copy(rows.at[b], out_hbm.at[pl.ds(base, W)],
                              wsem[b]).wait()

    fire(0, 0)
    fire(1, 1)
    fire(2, 2)

    def block(m, carry):
        for b in range(NBUF):
            g = m * NBUF + b
            drain(b)
            start_write(g, b)

            @pl.when(g >= 1)
            def _():
                wait_write((b + 3) % NBUF)

            @pl.when(g + 3 < NWIN)
            def _():
                fire(g + 3, (b + 3) % NBUF)
        return carry

    lax.fori_loop(0, NWIN // NBUF, block, 0)
    wait_write((NWIN - 1) % NBUF)


def kernel(x, table):
    idx = x.reshape(-1).astype(jnp.int32)
    out = _emb_lookup(table, idx)
    return out.reshape(x.shape + (table.shape[-1],))


# speedup vs baseline: 1.0147x; 1.0147x over previous
import functools

import jax
import jax.numpy as jnp
from jax import lax
from jax.experimental import pallas as pl
from jax.experimental.pallas import tpu as pltpu
from jax.experimental.pallas import tpu_sc as plsc

D = 64
N = 4096 * 200
NW = 32
PER_W = N // NW
W = 320
NWIN = PER_W // W
NBUF = 4
NG = W // 16

_mesh = plsc.VectorSubcoreMesh(core_axis_name="c", subcore_axis_name="s")


@functools.partial(
    pl.kernel,
    mesh=_mesh,
    out_type=jax.ShapeDtypeStruct((N, D), jnp.float32),
    scratch_types=[
        pltpu.VMEM((PER_W,), jnp.int32),
        pltpu.VMEM((NBUF, W, D), jnp.float32),
        [pltpu.SemaphoreType.DMA] * NBUF,
        [pltpu.SemaphoreType.DMA] * NBUF,
    ],
    compiler_params=pltpu.CompilerParams(use_tc_tiling_on_sc=False),
)
def _emb_lookup(table_hbm, idx_hbm, out_hbm, idx_all, rows, gsem, wsem):
    wid = lax.axis_index("s") * 2 + lax.axis_index("c")
    base = wid * PER_W
    pltpu.sync_copy(idx_hbm.at[pl.ds(base, PER_W)], idx_all)

    def fire(g, b):
        for i in range(NG):
            iv = idx_all[pl.ds(g * W + i * 16, 16)]
            pltpu.async_copy(table_hbm.at[iv], rows.at[b, pl.ds(i * 16, 16)],
                             gsem[b])

    def drain(b):
        for i in range(NG):
            pltpu.make_async_copy(
                table_hbm.at[idx_all[pl.ds(i * 16, 16)]],
                rows.at[b, pl.ds(i * 16, 16)], gsem[b]).wait()

    def start_write(g, b):
        return pltpu.async_copy(
            rows.at[b], out_hbm.at[pl.ds(base + g * W, W)], wsem[b])

    def wait_write(b):
        pltpu.make_async_copy(rows.at[b], out_hbm.at[pl.ds(base, W)],
                              wsem[b]).wait()

    fire(0, 0)
    fire(1, 1)
    fire(2, 2)

    def block(m, carry):
        for b in range(NBUF):
            g = m * NBUF + b
            drain(b)
            start_write(g, b)

            @pl.when(g >= 1)
            def _():
                wait_write((b + 3) % NBUF)

            @pl.when(g + 3 < NWIN)
            def _():
                fire(g + 3, (b + 3) % NBUF)
        return carry

    lax.fori_loop(0, NWIN // NBUF, block, 0)
    wait_write((NWIN - 1) % NBUF)


def kernel(x, table):
    idx = x.reshape(-1).astype(jnp.int32)
    out = _emb_lookup(table, idx)
    return out.reshape(x.shape + (table.shape[-1],))
